# Initial kernel scaffold; baseline (speedup 1.0000x reference)
import jax
import jax.numpy as jnp
from jax.experimental import pallas as pl


def kernel(sdc_traj_all, sdc_planning_gt, sdc_planning_gt_mask, bev_mask, bev_target):
    raise NotImplementedError("write your pallas kernel here")


# speedup vs baseline: 3.9332x; 3.9332x over previous
import jax
import jax.numpy as jnp
from jax.experimental import pallas as pl
from jax.experimental.pallas import tpu as pltpu

_H = 200
_W = 200
_NF = 6
_NL = 16


def _occ_loss_kernel(traj_ref, gmask_ref, mask_ref, out_ref, acc_ref):
    t = pl.program_id(0)

    @pl.when(t == 0)
    def _init():
        acc_ref[0] = 0.0
        acc_ref[1] = 0.0
        acc_ref[2] = 0.0

    m = mask_ref[:, 0]
    mx = jnp.max(m, axis=0)
    occ = (jax.nn.sigmoid(mx) > 0.1).astype(jnp.float32)
    acc_ref[0] += jnp.sum(occ)

    rr = jax.lax.broadcasted_iota(jnp.int32, (_H, _W), 0).astype(jnp.float32)
    cc = jax.lax.broadcasted_iota(jnp.int32, (_H, _W), 1).astype(jnp.float32)
    xg = jnp.trunc((cc - 100.0) * 0.5 + 0.25)
    yg = jnp.trunc((rr - 100.0) * 0.5 + 0.25)

    def add_future(i):
        px = traj_ref[i, 0]
        py = traj_ref[i, 1]
        g = gmask_ref[i]
        dx = px - xg
        dy = py - yg
        d2 = dx * dx + dy * dy
        keep = (d2 < 25.0).astype(jnp.float32)
        w = occ * keep
        cnt = jnp.sum(w)
        col = 0.5 * jnp.sum(jnp.exp(-0.5 * d2) * w) / 2.507
        valid_g = (cnt > 0).astype(jnp.float32) * g
        acc_ref[1] += col * valid_g
        acc_ref[2] += valid_g

    @pl.when(t > 0)
    def _mid():
        add_future(t - 1)

    @pl.when(t == _NF - 1)
    def _last():
        add_future(_NF - 1)

        ms = acc_ref[0]
        num = acc_ref[1]
        den = acc_ref[2]
        loss = jnp.where(den > 0.0, num / jnp.maximum(den, 1.0), 0.0)
        loss = jnp.where(ms == 0.0, 0.0, loss)
        out_ref[0] = loss


def kernel(sdc_traj_all, sdc_planning_gt, sdc_planning_gt_mask, bev_mask, bev_target):
    traj = sdc_traj_all[0].astype(jnp.float32)
    gmask = (sdc_planning_gt_mask[0] != 0).astype(jnp.float32)
    bev = bev_mask[0]

    out = pl.pallas_call(
        _occ_loss_kernel,
        grid=(_NF,),
        in_specs=[
            pl.BlockSpec(memory_space=pltpu.SMEM),
            pl.BlockSpec(memory_space=pltpu.SMEM),
            pl.BlockSpec((_NL, 1, _H, _W), lambda t: (0, t, 0, 0)),
        ],
        out_specs=pl.BlockSpec(memory_space=pltpu.SMEM),
        out_shape=jax.ShapeDtypeStruct((1,), jnp.float32),
        scratch_shapes=[pltpu.SMEM((4,), jnp.float32)],
    )(traj, gmask, bev)
    return out[0]
